# baseline (speedup 1.0000x reference)
import jax
import jax.numpy as jnp
from jax.experimental import pallas as pl
from jax.experimental.pallas import tpu as pltpu

_TOP_K = 8


def _gate_kernel(x_ref, w_ref, b_ref, idx_ref, wgt_ref):
    x = x_ref[...]
    w = w_ref[...]
    logits = jax.lax.dot_general(
        x, w, (((1,), (1,)), ((), ())), preferred_element_type=jnp.float32
    )
    scores = jax.nn.sigmoid(logits)
    biased = scores + b_ref[...]
    expert_ids = jax.lax.broadcasted_iota(jnp.int32, biased.shape, 1)
    neg_inf = jnp.float32(-jnp.inf)
    for k in range(_TOP_K):
        am = jnp.argmax(biased, axis=-1, keepdims=True)
        onehot = expert_ids == am
        s_k = jnp.sum(jnp.where(onehot, scores, 0.0), axis=-1, keepdims=True)
        idx_ref[:, k : k + 1] = am.astype(jnp.int32)
        wgt_ref[:, k : k + 1] = s_k
        biased = jnp.where(onehot, neg_inf, biased)
    sel = wgt_ref[...]
    inv = 1.0 / (jnp.sum(sel, axis=-1, keepdims=True) + 1e-20)
    wgt_ref[...] = sel * inv


def kernel(x, gate_weight, bias):
    n_tokens, d_model = x.shape
    n_experts = gate_weight.shape[0]
    block_tokens = 1024
    grid = (n_tokens // block_tokens,)
    bias2d = bias.reshape(1, n_experts)
    idx, wgt = pl.pallas_call(
        _gate_kernel,
        grid=grid,
        in_specs=[
            pl.BlockSpec((block_tokens, d_model), lambda i: (i, 0)),
            pl.BlockSpec((n_experts, d_model), lambda i: (0, 0)),
            pl.BlockSpec((1, n_experts), lambda i: (0, 0)),
        ],
        out_specs=[
            pl.BlockSpec((block_tokens, _TOP_K), lambda i: (i, 0)),
            pl.BlockSpec((block_tokens, _TOP_K), lambda i: (i, 0)),
        ],
        out_shape=[
            jax.ShapeDtypeStruct((n_tokens, _TOP_K), jnp.int32),
            jax.ShapeDtypeStruct((n_tokens, _TOP_K), jnp.float32),
        ],
        compiler_params=pltpu.CompilerParams(
            dimension_semantics=("parallel",),
        ),
    )(x, gate_weight, bias2d)
    return idx, wgt


# speedup vs baseline: 1.0335x; 1.0335x over previous
import jax
import jax.numpy as jnp
from jax.experimental import pallas as pl
from jax.experimental.pallas import tpu as pltpu

_TOP_K = 8


def _gate_kernel(x_ref, w_ref, b_ref, idx_ref, wgt_ref):
    x = x_ref[...]
    w = w_ref[...]
    logits = jax.lax.dot_general(
        x, w, (((1,), (1,)), ((), ())), preferred_element_type=jnp.float32
    )
    scores = jax.nn.sigmoid(logits)
    biased = scores + b_ref[...]
    expert_ids = jax.lax.broadcasted_iota(jnp.int32, biased.shape, 1)
    neg_inf = jnp.float32(-jnp.inf)
    for k in range(_TOP_K):
        am = jnp.argmax(biased, axis=-1, keepdims=True)
        onehot = expert_ids == am
        s_k = jnp.sum(jnp.where(onehot, scores, 0.0), axis=-1, keepdims=True)
        idx_ref[:, k : k + 1] = am.astype(jnp.int32)
        wgt_ref[:, k : k + 1] = s_k
        biased = jnp.where(onehot, neg_inf, biased)
    sel = wgt_ref[...]
    inv = 1.0 / (jnp.sum(sel, axis=-1, keepdims=True) + 1e-20)
    wgt_ref[...] = sel * inv


def kernel(x, gate_weight, bias):
    n_tokens, d_model = x.shape
    n_experts = gate_weight.shape[0]
    block_tokens = 2048
    grid = (n_tokens // block_tokens,)
    bias2d = bias.reshape(1, n_experts)
    idx, wgt = pl.pallas_call(
        _gate_kernel,
        grid=grid,
        in_specs=[
            pl.BlockSpec((block_tokens, d_model), lambda i: (i, 0)),
            pl.BlockSpec((n_experts, d_model), lambda i: (0, 0)),
            pl.BlockSpec((1, n_experts), lambda i: (0, 0)),
        ],
        out_specs=[
            pl.BlockSpec((block_tokens, _TOP_K), lambda i: (i, 0)),
            pl.BlockSpec((block_tokens, _TOP_K), lambda i: (i, 0)),
        ],
        out_shape=[
            jax.ShapeDtypeStruct((n_tokens, _TOP_K), jnp.int32),
            jax.ShapeDtypeStruct((n_tokens, _TOP_K), jnp.float32),
        ],
        compiler_params=pltpu.CompilerParams(
            dimension_semantics=("parallel",),
        ),
    )(x, gate_weight, bias2d)
    return idx, wgt
